# Initial kernel scaffold; baseline (speedup 1.0000x reference)
import jax
import jax.numpy as jnp
from jax.experimental import pallas as pl


def kernel(x, w0, w1, w2, w3, wp, g0, b0, m0, v0, g1, b1, m1, v1, g2, b2, m2, v2, g3, b3, m3, v3, gp, bp, mp, vp):
    raise NotImplementedError("write your pallas kernel here")


# speedup vs baseline: 1.2429x; 1.2429x over previous
import jax
import jax.numpy as jnp
from jax import lax
from jax.experimental import pallas as pl
from jax.experimental.pallas import tpu as pltpu

_EPS = 1e-5
_RATES = (6, 12, 18)
_H = 64
_HW = _H * _H
_PAD = 1280
_XPW = _HW + 2 * _PAD


def _taps():
    out = {0: [(0, 0, 0)]}
    t = 1
    for bi, d in enumerate(_RATES, start=1):
        lst = []
        for kh in (-1, 0, 1):
            for kw in (-1, 0, 1):
                lst.append((t, kh * d * _H + kw * d, kw * d))
                t += 1
        out[bi] = lst
    return out


_TAPS = _taps()


def _asppup_kernel(xp_ref, wt_ref, bb_ref, wp_ref, bp_ref, o_ref):
    col = lax.broadcasted_iota(jnp.int32, (1, _HW), 1) % _H
    wp = wp_ref[...]
    for br in range(4):
        acc = None
        for (t, s, cw) in _TAPS[br]:
            xs = xp_ref[0, :, _PAD + s:_PAD + s + _HW]
            y = jnp.dot(wt_ref[t], xs, preferred_element_type=jnp.float32)
            if cw > 0:
                y = jnp.where(col < _H - cw, y, 0.0)
            elif cw < 0:
                y = jnp.where(col >= -cw, y, 0.0)
            acc = y if acc is None else acc + y
        a = jnp.maximum(acc + bb_ref[br, :, 0:1], 0.0)
        z = jnp.dot(wp, a, preferred_element_type=jnp.float32)
        o_ref[0, br] = jnp.maximum(z + bp_ref[:, 0:1], 0.0)


def kernel(x, w0, w1, w2, w3, wp,
           g0, b0, m0, v0, g1, b1, m1, v1,
           g2, b2, m2, v2, g3, b3, m3, v3,
           gp, bp, mp, vp):
    B, Cin, H, W = x.shape
    Cout = w0.shape[0]

    x2 = x.reshape(B, Cin, H * W)
    xp = jnp.pad(x2, ((0, 0), (0, 0), (_PAD, _PAD)))

    def fold(w, g, b, m, v):
        s = g * lax.rsqrt(v + _EPS)
        return w * s[:, None, None, None], b - m * s

    w0f, bias0 = fold(w0, g0, b0, m0, v0)
    rows = [w0f[:, :, 0, 0]]
    biases = [bias0]
    for w, g, b, m, v in ((w1, g1, b1, m1, v1),
                          (w2, g2, b2, m2, v2),
                          (w3, g3, b3, m3, v3)):
        wf, bi = fold(w, g, b, m, v)
        for kh in range(3):
            for kw in range(3):
                rows.append(wf[:, :, kh, kw])
        biases.append(bi)
    wt = jnp.stack(rows)
    bb = jnp.broadcast_to(jnp.stack(biases)[:, :, None], (4, Cout, 128))
    sp = gp * lax.rsqrt(vp + _EPS)
    wpf = wp[:, :, 0, 0] * sp[:, None]
    bpf = jnp.broadcast_to((bp - mp * sp)[:, None], (Cout, 128))

    out = pl.pallas_call(
        _asppup_kernel,
        grid=(B,),
        in_specs=[
            pl.BlockSpec((1, Cin, _XPW), lambda b: (b, 0, 0)),
            pl.BlockSpec((28, Cout, Cin), lambda b: (0, 0, 0)),
            pl.BlockSpec((4, Cout, 128), lambda b: (0, 0, 0)),
            pl.BlockSpec((Cout, Cout), lambda b: (0, 0)),
            pl.BlockSpec((Cout, 128), lambda b: (0, 0)),
        ],
        out_specs=pl.BlockSpec((1, 4, Cout, _HW), lambda b: (b, 0, 0, 0)),
        out_shape=jax.ShapeDtypeStruct((B, 4, Cout, _HW), jnp.float32),
        compiler_params=pltpu.CompilerParams(
            dimension_semantics=("parallel",),
            vmem_limit_bytes=52 * 1024 * 1024,
        ),
    )(xp, wt, bb, wpf, bpf)

    z = out.reshape(B, 2, 2, Cout, H, W).transpose(0, 3, 4, 1, 5, 2)
    return z.reshape(B, Cout, 2 * H, 2 * W)


# speedup vs baseline: 1.4933x; 1.2015x over previous
import jax
import jax.numpy as jnp
from jax import lax
from jax.experimental import pallas as pl
from jax.experimental.pallas import tpu as pltpu

_EPS = 1e-5
_RATES = (6, 12, 18)
_H = 64
_HW = _H * _H
_PAD = 1280
_XPW = _HW + 2 * _PAD


def _taps():
    out = {0: [(0, 0, 0)]}
    t = 1
    for bi, d in enumerate(_RATES, start=1):
        lst = []
        for kh in (-1, 0, 1):
            for kw in (-1, 0, 1):
                lst.append((t, kh * d * _H + kw * d, kw * d))
                t += 1
        out[bi] = lst
    return out


_TAPS = _taps()


def _asppup_kernel(x_ref, wt_ref, bb_ref, wp_ref, bp_ref, o_ref, xs_ref):
    cin = x_ref.shape[1]
    xs_ref[:, :_PAD] = jnp.zeros((cin, _PAD), jnp.bfloat16)
    xs_ref[:, _PAD + _HW:] = jnp.zeros((cin, _PAD), jnp.bfloat16)
    xs_ref[:, _PAD:_PAD + _HW] = x_ref[0].astype(jnp.bfloat16)

    col = lax.broadcasted_iota(jnp.int32, (1, _HW), 1) % _H
    wp = wp_ref[...]
    for br in range(4):
        acc = None
        for (t, s, cw) in _TAPS[br]:
            xs = xs_ref[:, _PAD + s:_PAD + s + _HW]
            y = jnp.dot(wt_ref[t], xs, preferred_element_type=jnp.float32)
            if cw > 0:
                y = jnp.where(col < _H - cw, y, 0.0)
            elif cw < 0:
                y = jnp.where(col >= -cw, y, 0.0)
            acc = y if acc is None else acc + y
        a = jnp.maximum(acc + bb_ref[br, :, 0:1], 0.0)
        z = jnp.dot(wp, a.astype(jnp.bfloat16), preferred_element_type=jnp.float32)
        o_ref[0, br] = jnp.maximum(z + bp_ref[:, 0:1], 0.0)


def kernel(x, w0, w1, w2, w3, wp,
           g0, b0, m0, v0, g1, b1, m1, v1,
           g2, b2, m2, v2, g3, b3, m3, v3,
           gp, bp, mp, vp):
    B, Cin, H, W = x.shape
    Cout = w0.shape[0]

    x2 = x.reshape(B, Cin, H * W)

    def fold(w, g, b, m, v):
        s = g * lax.rsqrt(v + _EPS)
        return w * s[:, None, None, None], b - m * s

    w0f, bias0 = fold(w0, g0, b0, m0, v0)
    rows = [w0f[:, :, 0, 0]]
    biases = [bias0]
    for w, g, b, m, v in ((w1, g1, b1, m1, v1),
                          (w2, g2, b2, m2, v2),
                          (w3, g3, b3, m3, v3)):
        wf, bi = fold(w, g, b, m, v)
        for kh in range(3):
            for kw in range(3):
                rows.append(wf[:, :, kh, kw])
        biases.append(bi)
    wt = jnp.stack(rows).astype(jnp.bfloat16)
    bb = jnp.broadcast_to(jnp.stack(biases)[:, :, None], (4, Cout, 128))
    sp = gp * lax.rsqrt(vp + _EPS)
    wpf = (wp[:, :, 0, 0] * sp[:, None]).astype(jnp.bfloat16)
    bpf = jnp.broadcast_to((bp - mp * sp)[:, None], (Cout, 128))

    out = pl.pallas_call(
        _asppup_kernel,
        grid=(B,),
        in_specs=[
            pl.BlockSpec((1, Cin, _HW), lambda b: (b, 0, 0)),
            pl.BlockSpec((28, Cout, Cin), lambda b: (0, 0, 0)),
            pl.BlockSpec((4, Cout, 128), lambda b: (0, 0, 0)),
            pl.BlockSpec((Cout, Cout), lambda b: (0, 0)),
            pl.BlockSpec((Cout, 128), lambda b: (0, 0)),
        ],
        out_specs=pl.BlockSpec((1, 4, Cout, _HW), lambda b: (b, 0, 0, 0)),
        out_shape=jax.ShapeDtypeStruct((B, 4, Cout, _HW), jnp.float32),
        scratch_shapes=[pltpu.VMEM((Cin, _XPW), jnp.bfloat16)],
        compiler_params=pltpu.CompilerParams(
            dimension_semantics=("parallel",),
            vmem_limit_bytes=52 * 1024 * 1024,
        ),
    )(x2, wt, bb, wpf, bpf)

    z = out.reshape(B, 2, 2, Cout, H, W).transpose(0, 3, 4, 1, 5, 2)
    return z.reshape(B, Cout, 2 * H, 2 * W)


# speedup vs baseline: 1.4942x; 1.0006x over previous
import jax
import jax.numpy as jnp
from jax import lax
from jax.experimental import pallas as pl
from jax.experimental.pallas import tpu as pltpu

_EPS = 1e-5
_RATES = (6, 12, 18)
_H = 64
_HW = _H * _H
_PAD = 1280
_XPW = _HW + 2 * _PAD


def _taps():
    out = {0: [(0, 0, 0)]}
    t = 1
    for bi, d in enumerate(_RATES, start=1):
        lst = []
        for kh in (-1, 0, 1):
            for kw in (-1, 0, 1):
                lst.append((t, kh * d * _H + kw * d, kw * d))
                t += 1
        out[bi] = lst
    return out


_TAPS = _taps()


def _asppup_kernel(x_ref, wt_ref, bb_ref, wp_ref, bp_ref, o_ref, xs_ref):
    cin = x_ref.shape[1]
    xs_ref[:, :_PAD] = jnp.zeros((cin, _PAD), jnp.bfloat16)
    xs_ref[:, _PAD + _HW:] = jnp.zeros((cin, _PAD), jnp.bfloat16)
    xs_ref[:, _PAD:_PAD + _HW] = x_ref[0].astype(jnp.bfloat16)

    col = lax.broadcasted_iota(jnp.int32, (1, _HW), 1) % _H
    wp = wp_ref[...]
    for br in range(4):
        acc = None
        for (t, s, cw) in _TAPS[br]:
            xs = xs_ref[:, _PAD + s:_PAD + s + _HW]
            y = jnp.dot(wt_ref[t], xs, preferred_element_type=jnp.float32)
            if cw > 0:
                y = jnp.where(col < _H - cw, y, 0.0)
            elif cw < 0:
                y = jnp.where(col >= -cw, y, 0.0)
            acc = y if acc is None else acc + y
        a = jnp.maximum(acc + bb_ref[br, :, 0:1], 0.0)
        z = jnp.dot(wp, a.astype(jnp.bfloat16), preferred_element_type=jnp.float32)
        o_ref[0, br] = jnp.maximum(z + bp_ref[:, 0:1], 0.0)


def kernel(x, w0, w1, w2, w3, wp,
           g0, b0, m0, v0, g1, b1, m1, v1,
           g2, b2, m2, v2, g3, b3, m3, v3,
           gp, bp, mp, vp):
    B, Cin, H, W = x.shape
    Cout = w0.shape[0]

    x2 = x.reshape(B, Cin, H * W)

    def fold(w, g, b, m, v):
        s = g * lax.rsqrt(v + _EPS)
        return w * s[:, None, None, None], b - m * s

    w0f, bias0 = fold(w0, g0, b0, m0, v0)
    rows = [w0f[:, :, 0, 0]]
    biases = [bias0]
    for w, g, b, m, v in ((w1, g1, b1, m1, v1),
                          (w2, g2, b2, m2, v2),
                          (w3, g3, b3, m3, v3)):
        wf, bi = fold(w, g, b, m, v)
        for kh in range(3):
            for kw in range(3):
                rows.append(wf[:, :, kh, kw])
        biases.append(bi)
    wt = jnp.stack(rows).astype(jnp.bfloat16)
    bb = jnp.broadcast_to(jnp.stack(biases)[:, :, None], (4, Cout, 128))
    sp = gp * lax.rsqrt(vp + _EPS)
    wpf = (wp[:, :, 0, 0] * sp[:, None]).astype(jnp.bfloat16)
    bpf = jnp.broadcast_to((bp - mp * sp)[:, None], (Cout, 128))

    out = pl.pallas_call(
        _asppup_kernel,
        grid=(B,),
        in_specs=[
            pl.BlockSpec((1, Cin, _HW), lambda b: (b, 0, 0)),
            pl.BlockSpec((28, Cout, Cin), lambda b: (0, 0, 0)),
            pl.BlockSpec((4, Cout, 128), lambda b: (0, 0, 0)),
            pl.BlockSpec((Cout, Cout), lambda b: (0, 0)),
            pl.BlockSpec((Cout, 128), lambda b: (0, 0)),
        ],
        out_specs=pl.BlockSpec((1, 4, Cout, _HW), lambda b: (b, 0, 0, 0)),
        out_shape=jax.ShapeDtypeStruct((B, 4, Cout, _HW), jnp.float32),
        scratch_shapes=[pltpu.VMEM((Cin, _XPW), jnp.bfloat16)],
        compiler_params=pltpu.CompilerParams(
            dimension_semantics=("arbitrary",),
            vmem_limit_bytes=52 * 1024 * 1024,
        ),
    )(x2, wt, bb, wpf, bpf)

    z = out.reshape(B, 2, 2, Cout, H, W).transpose(0, 3, 4, 1, 5, 2)
    return z.reshape(B, Cout, 2 * H, 2 * W)
